# baseline (speedup 1.0000x reference)
import functools

import jax
import jax.numpy as jnp
from jax import lax
from jax.experimental import pallas as pl
from jax.experimental.pallas import tpu as pltpu
from jax.experimental.pallas import tpu_sc as plsc

N_USERS = 25000
N_ITEMS = 25000
EMBED = 64
NLAYERS = 3
NPAD = 25088
STRIPE = NPAD // 16
N_EDGES = 800000
B = 64
SUP = 8
RING = 4
E_PAD = 819200
NBATCH = E_PAD // B
NB_TILE = NBATCH // 16
NSUP = NB_TILE // SUP


def _layer_body(rows_hbm, cols_hbm, v16_hbm, tu_hbm, ti_hbm, zrow_hbm,
                out_u, out_i,
                srcs, dsts, v16s, rows0, rows1, rows2, rows3, acc,
                g0, g1, g2, g3, s0, s1, s2, s3):
    c = lax.axis_index("c")
    s = lax.axis_index("s")
    rows_bufs = (rows0, rows1, rows2, rows3)
    gsems = (g0, g1, g2, g3)
    ssems = (s0, s1, s2, s3)

    def do_spmm(srcidx_hbm, dstidx_hbm, table_hbm, out_hbm):
        pltpu.sync_copy(zrow_hbm, acc.at[pl.ds(s * STRIPE, STRIPE)])
        plsc.subcore_barrier()
        b0 = s * NB_TILE

        def super_body(k, carry):
            kb = b0 + k * SUP
            pltpu.sync_copy(srcidx_hbm.at[pl.ds(kb, SUP)], srcs)
            pltpu.sync_copy(dstidx_hbm.at[pl.ds(kb, SUP)], dsts)
            pltpu.sync_copy(v16_hbm.at[pl.ds(kb, SUP)], v16s)
            gd = [None] * SUP
            sd = [None] * SUP
            gd[0] = pltpu.async_copy(table_hbm.at[srcs.at[0]], rows0, g0)
            for jj in range(SUP):
                p = jj % RING
                if jj + 1 < SUP:
                    q = (jj + 1) % RING
                    if jj + 1 >= RING:
                        sd[jj + 1 - RING].wait()
                    gd[jj + 1] = pltpu.async_copy(
                        table_hbm.at[srcs.at[jj + 1]], rows_bufs[q], gsems[q])
                gd[jj].wait()
                rv = rows_bufs[p]

                @plsc.parallel_loop(0, B, unroll=4)
                def _(e):
                    vb = v16s[jj, pl.ds(e * 16, 16)]
                    for u in range(EMBED // 16):
                        rv[e, pl.ds(u * 16, 16)] = rv[e, pl.ds(u * 16, 16)] * vb

                sd[jj] = pltpu.async_copy(rv, acc.at[dsts.at[jj]], ssems[p],
                                          add=True)
            for jj in range(SUP - RING, SUP):
                sd[jj].wait()
            return carry

        lax.fori_loop(0, NSUP, super_body, 0)
        plsc.subcore_barrier()
        pltpu.sync_copy(acc.at[pl.ds(s * STRIPE, STRIPE)],
                        out_hbm.at[pl.ds(s * STRIPE, STRIPE)])

    @pl.when(c == 0)
    def _():
        do_spmm(cols_hbm, rows_hbm, ti_hbm, out_u)

    @pl.when(c == 1)
    def _():
        do_spmm(rows_hbm, cols_hbm, tu_hbm, out_i)


_layer = pl.kernel(
    _layer_body,
    out_type=[jax.ShapeDtypeStruct((NPAD, EMBED), jnp.float32)] * 2,
    mesh=plsc.VectorSubcoreMesh(core_axis_name="c", subcore_axis_name="s"),
    compiler_params=pltpu.CompilerParams(use_tc_tiling_on_sc=False),
    scratch_types=[
        pltpu.VMEM((SUP, B), jnp.int32),
        pltpu.VMEM((SUP, B), jnp.int32),
        pltpu.VMEM((SUP, B * 16), jnp.float32),
        pltpu.VMEM((B, EMBED), jnp.float32),
        pltpu.VMEM((B, EMBED), jnp.float32),
        pltpu.VMEM((B, EMBED), jnp.float32),
        pltpu.VMEM((B, EMBED), jnp.float32),
        pltpu.VMEM_SHARED((NPAD, EMBED), jnp.float32),
        pltpu.SemaphoreType.DMA,
        pltpu.SemaphoreType.DMA,
        pltpu.SemaphoreType.DMA,
        pltpu.SemaphoreType.DMA,
        pltpu.SemaphoreType.DMA,
        pltpu.SemaphoreType.DMA,
        pltpu.SemaphoreType.DMA,
        pltpu.SemaphoreType.DMA,
    ],
)


def _tail_body(u0, u1, u2, u3, i0, i1, i2, i3, ums, umap, vms, imap,
               out_u, out_i, out_us, out_is):
    out_u[...] = (u0[...] + u1[...] + u2[...] + u3[...]) * 0.25
    out_i[...] = (i0[...] + i1[...] + i2[...] + i3[...]) * 0.25
    out_us[...] = jnp.dot(ums[...], umap[...], preferred_element_type=jnp.float32)
    out_is[...] = jnp.dot(vms[...], imap[...], preferred_element_type=jnp.float32)


def _tail(u_list, i_list, ums, umap, vms, imap):
    blk = 1000
    grid = (N_USERS // blk,)
    row_spec = pl.BlockSpec((blk, EMBED), lambda i: (i, 0))
    map_spec = pl.BlockSpec((64, EMBED), lambda i: (0, 0))
    return pl.pallas_call(
        _tail_body,
        grid=grid,
        in_specs=[row_spec] * 8 + [row_spec, map_spec, row_spec, map_spec],
        out_specs=[row_spec] * 4,
        out_shape=[jax.ShapeDtypeStruct((N_USERS, EMBED), jnp.float32)] * 4,
    )(*u_list, *i_list, ums, umap, vms, imap)


def kernel(edge_index, edge_vals, user_preference, item_preference,
           user_map, item_map, U_mul_S, V_mul_S):
    npad_e = E_PAD - N_EDGES
    rows = jnp.concatenate(
        [edge_index[0], jnp.full((npad_e,), N_USERS, jnp.int32)]).reshape(NBATCH, B)
    cols = jnp.concatenate(
        [edge_index[1], jnp.full((npad_e,), N_ITEMS, jnp.int32)]).reshape(NBATCH, B)
    vals_flat = jnp.concatenate([edge_vals, jnp.zeros((npad_e,), jnp.float32)])
    v16 = jnp.repeat(vals_flat[:, None], 16, axis=1).reshape(NBATCH, B * 16)
    pad = ((0, NPAD - N_USERS), (0, 0))
    u_list = [jnp.pad(user_preference, pad)]
    i_list = [jnp.pad(item_preference, pad)]
    zrow = jnp.zeros((STRIPE, EMBED), jnp.float32)
    for _ in range(NLAYERS):
        u_next, i_next = _layer(rows, cols, v16, u_list[-1], i_list[-1], zrow)
        u_list.append(u_next)
        i_list.append(i_next)
    pu, pi, su, si = _tail(u_list, i_list, U_mul_S, user_map, V_mul_S, item_map)
    return jnp.stack([pu, pi, su, si], axis=0)


# speedup vs baseline: 1.7614x; 1.7614x over previous
import functools

import jax
import jax.numpy as jnp
from jax import lax
from jax.experimental import pallas as pl
from jax.experimental.pallas import tpu as pltpu
from jax.experimental.pallas import tpu_sc as plsc

N_USERS = 25000
N_ITEMS = 25000
EMBED = 64
NLAYERS = 3
NPAD = 25088
STRIPE = NPAD // 16
N_EDGES = 800000
B = 192
SUP = 4
RING = 2
E_PAD = 811008
NBATCH = E_PAD // B
NB_TILE = NBATCH // 16
NSUP = NB_TILE // SUP


def _layer_body(rows_hbm, cols_hbm, vals_hbm, tu_hbm, ti_hbm, zrow_hbm,
                out_u, out_i,
                srcs, dsts, valss, rows0, rows1, acc,
                g0, g1, s0, s1):
    c = lax.axis_index("c")
    s = lax.axis_index("s")
    rows_bufs = (rows0, rows1)
    gsems = (g0, g1)
    ssems = (s0, s1)

    def do_spmm(srcidx_hbm, dstidx_hbm, table_hbm, out_hbm):
        pltpu.sync_copy(zrow_hbm, acc.at[pl.ds(s * STRIPE, STRIPE)])
        plsc.subcore_barrier()
        b0 = s * NB_TILE

        def super_body(k, carry):
            kb = b0 + k * SUP
            pltpu.sync_copy(srcidx_hbm.at[pl.ds(kb, SUP)], srcs)
            pltpu.sync_copy(dstidx_hbm.at[pl.ds(kb, SUP)], dsts)
            pltpu.sync_copy(vals_hbm.at[pl.ds(kb, SUP)], valss)
            gd = [None] * SUP
            sd = [None] * SUP
            gd[0] = pltpu.async_copy(table_hbm.at[srcs.at[0]], rows0, g0)
            for jj in range(SUP):
                p = jj % RING
                if jj + 1 < SUP:
                    q = (jj + 1) % RING
                    if jj + 1 >= RING:
                        sd[jj + 1 - RING].wait()
                    gd[jj + 1] = pltpu.async_copy(
                        table_hbm.at[srcs.at[jj + 1]], rows_bufs[q], gsems[q])
                gd[jj].wait()
                rv = rows_bufs[p]

                @plsc.parallel_loop(0, B // 16, unroll=2)
                def _(g):
                    vv = valss[jj, pl.ds(g * 16, 16)]
                    for t in range(16):
                        v = vv[t]
                        e = g * 16 + t
                        for u in range(EMBED // 16):
                            rv[e, pl.ds(u * 16, 16)] = rv[e, pl.ds(u * 16, 16)] * v

                sd[jj] = pltpu.async_copy(rv, acc.at[dsts.at[jj]], ssems[p],
                                          add=True)
            for jj in range(SUP - RING, SUP):
                sd[jj].wait()
            return carry

        lax.fori_loop(0, NSUP, super_body, 0)
        plsc.subcore_barrier()
        pltpu.sync_copy(acc.at[pl.ds(s * STRIPE, STRIPE)],
                        out_hbm.at[pl.ds(s * STRIPE, STRIPE)])

    @pl.when(c == 0)
    def _():
        do_spmm(cols_hbm, rows_hbm, ti_hbm, out_u)

    @pl.when(c == 1)
    def _():
        do_spmm(rows_hbm, cols_hbm, tu_hbm, out_i)


_layer = pl.kernel(
    _layer_body,
    out_type=[jax.ShapeDtypeStruct((NPAD, EMBED), jnp.float32)] * 2,
    mesh=plsc.VectorSubcoreMesh(core_axis_name="c", subcore_axis_name="s"),
    compiler_params=pltpu.CompilerParams(use_tc_tiling_on_sc=False),
    scratch_types=[
        pltpu.VMEM((SUP, B), jnp.int32),
        pltpu.VMEM((SUP, B), jnp.int32),
        pltpu.VMEM((SUP, B), jnp.float32),
        pltpu.VMEM((B, EMBED), jnp.float32),
        pltpu.VMEM((B, EMBED), jnp.float32),
        pltpu.VMEM_SHARED((NPAD, EMBED), jnp.float32),
        pltpu.SemaphoreType.DMA,
        pltpu.SemaphoreType.DMA,
        pltpu.SemaphoreType.DMA,
        pltpu.SemaphoreType.DMA,
    ],
)


def _tail_body(u0, u1, u2, u3, i0, i1, i2, i3, ums, umap, vms, imap,
               out_u, out_i, out_us, out_is):
    out_u[...] = (u0[...] + u1[...] + u2[...] + u3[...]) * 0.25
    out_i[...] = (i0[...] + i1[...] + i2[...] + i3[...]) * 0.25
    out_us[...] = jnp.dot(ums[...], umap[...], preferred_element_type=jnp.float32)
    out_is[...] = jnp.dot(vms[...], imap[...], preferred_element_type=jnp.float32)


def _tail(u_list, i_list, ums, umap, vms, imap):
    blk = 1000
    grid = (N_USERS // blk,)
    row_spec = pl.BlockSpec((blk, EMBED), lambda i: (i, 0))
    map_spec = pl.BlockSpec((64, EMBED), lambda i: (0, 0))
    return pl.pallas_call(
        _tail_body,
        grid=grid,
        in_specs=[row_spec] * 8 + [row_spec, map_spec, row_spec, map_spec],
        out_specs=[row_spec] * 4,
        out_shape=[jax.ShapeDtypeStruct((N_USERS, EMBED), jnp.float32)] * 4,
    )(*u_list, *i_list, ums, umap, vms, imap)


def kernel(edge_index, edge_vals, user_preference, item_preference,
           user_map, item_map, U_mul_S, V_mul_S):
    npad_e = E_PAD - N_EDGES
    rows = jnp.concatenate(
        [edge_index[0], jnp.full((npad_e,), N_USERS, jnp.int32)]).reshape(NBATCH, B)
    cols = jnp.concatenate(
        [edge_index[1], jnp.full((npad_e,), N_ITEMS, jnp.int32)]).reshape(NBATCH, B)
    vals = jnp.concatenate(
        [edge_vals, jnp.zeros((npad_e,), jnp.float32)]).reshape(NBATCH, B)
    pad = ((0, NPAD - N_USERS), (0, 0))
    u_list = [jnp.pad(user_preference, pad)]
    i_list = [jnp.pad(item_preference, pad)]
    zrow = jnp.zeros((STRIPE, EMBED), jnp.float32)
    for _ in range(NLAYERS):
        u_next, i_next = _layer(rows, cols, vals, u_list[-1], i_list[-1], zrow)
        u_list.append(u_next)
        i_list.append(i_next)
    pu, pi, su, si = _tail(u_list, i_list, U_mul_S, user_map, V_mul_S, item_map)
    return jnp.stack([pu, pi, su, si], axis=0)
